# baseline (speedup 1.0000x reference)
import jax
import jax.numpy as jnp
import numpy as np
from jax import lax
from jax.experimental import pallas as pl
from jax.experimental.pallas import tpu as pltpu
from jax.experimental.pallas import tpu_sc as plsc

VOCAB, F = 30522, 64
NT, R = 100000, 8
B, L = 4096, 200
T = B * L
NC, NS, LANES = 2, 16, 16
NW = NC * NS
TPW = T // NW
C = 400
BPC = C // L
G = TPW // C
CR = C * R
KPJ = C // LANES

_PERM = np.concatenate(
    [32 * g + np.arange(32).reshape(2, 16).T.reshape(-1) for g in range(F // 32)]
)


def _body(emb, res, bidx, out, bidx2, ids2, fidx2, acc2, fout2, sem_ids,
          sem_add, sem_out):
    cid = lax.axis_index("c")
    sid = lax.axis_index("s")
    wid = sid * NC + cid
    base = wid * TPW

    iota = lax.iota(jnp.int32, LANES)
    zv = jnp.zeros((2 * LANES,), jnp.bfloat16)
    himask = jnp.full((LANES,), -65536, jnp.int32)
    sh16 = jnp.full((LANES,), 16, jnp.int32)

    def front(g, p):
        tok0 = base + g * C
        pltpu.sync_copy(bidx.at[pl.ds(tok0, C)], bidx2.at[p])
        pltpu.async_copy(res.at[bidx2.at[p]], ids2.at[p], sem_ids)

    def wait_ids(p):
        pltpu.make_async_copy(res.at[bidx2.at[p]], ids2.at[p], sem_ids).wait()

    def build(p):
        for k in range(R * KPJ):
            j = k // KPJ
            rows = iota + (k % KPJ) * LANES
            cols = jnp.full((LANES,), j, jnp.int32)
            ids16 = plsc.load_gather(ids2.at[p], [rows, cols])
            fidx2[p, pl.ds(k * LANES, LANES)] = jnp.where(ids16 == 0, -1, ids16)

    def zero_acc(p):
        def zb(t, _):
            for jj in range(F // (2 * LANES)):
                acc2[p, t, pl.ds(jj * 2 * LANES, 2 * LANES)] = zv
            return ()

        lax.fori_loop(0, C, zb, ())

    def widen(p):
        def wb(q, _):
            for par in range(2):
                for grp in range(F // (2 * LANES)):
                    v = acc2[p, 2 * q + par, pl.ds(grp * 2 * LANES, 2 * LANES)]
                    w = plsc.bitcast(v, jnp.int32)
                    lo = plsc.bitcast(lax.shift_left(w, sh16), jnp.float32)
                    hi = plsc.bitcast(lax.bitwise_and(w, himask), jnp.float32)
                    fout2[p, q, pl.ds(par * F + grp * 2 * LANES, LANES)] = lo
                    fout2[p, q, pl.ds(par * F + grp * 2 * LANES + LANES, LANES)] = hi
            return ()

        lax.fori_loop(0, C // 2, wb, ())

    def fire_adds(p):
        descs = []
        for j in range(R):
            idx = plsc.Indices(
                fidx2.at[p].at[pl.ds(j * C, C)], ignored_value=-1
            )
            descs.append(
                pltpu.async_copy(emb.at[idx], acc2.at[p], sem_add, add=True)
            )
        return descs

    def out_issue(g, p):
        row0 = (base + g * C) // 2
        pltpu.async_copy(
            fout2.at[p], out.at[pl.ds(row0, C // 2), :], sem_out
        )

    def wait_out(g, p):
        row0 = (base + g * C) // 2
        pltpu.make_async_copy(
            fout2.at[p], out.at[pl.ds(row0, C // 2), :], sem_out
        ).wait()

    front(0, 0)
    wait_ids(0)
    build(0)
    zero_acc(0)

    def loop_body(i, _):
        for ph in range(2):
            g = 2 * i + ph
            p = ph

            @pl.when(g + 1 < G)
            def _next_front():
                front(g + 1, 1 - p)

            descs = fire_adds(p)

            @pl.when(g >= 3)
            def _reclaim_fout():
                wait_out(g - 3, 1 - p)

            @pl.when(g >= 1)
            def _drain_prev():
                widen(1 - p)
                out_issue(g - 1, 1 - p)

            @pl.when(g + 1 < G)
            def _next_prep():
                wait_ids(1 - p)
                build(1 - p)
                zero_acc(1 - p)

            for d in descs:
                d.wait()
        return ()

    lax.fori_loop(0, G // 2, loop_body, ())
    wait_out(G - 3, 1)
    widen(1)
    out_issue(G - 1, 1)
    wait_out(G - 2, 0)
    wait_out(G - 1, 1)


@jax.jit
def kernel(embedding, reservoir_encoded, base_indices):
    mesh = plsc.VectorSubcoreMesh(core_axis_name="c", subcore_axis_name="s")
    run = pl.kernel(
        _body,
        out_type=jax.ShapeDtypeStruct((T // 2, 2 * F), jnp.float32),
        mesh=mesh,
        compiler_params=pltpu.CompilerParams(
            needs_layout_passes=False, use_tc_tiling_on_sc=False
        ),
        scratch_types=[
            pltpu.VMEM((2, C), jnp.int32),
            pltpu.VMEM((2, C, R), jnp.int32),
            pltpu.VMEM((2, CR), jnp.int32),
            pltpu.VMEM((2, C, F), jnp.bfloat16),
            pltpu.VMEM((2, C // 2, 2 * F), jnp.float32),
            pltpu.SemaphoreType.DMA,
            pltpu.SemaphoreType.DMA,
            pltpu.SemaphoreType.DMA,
        ],
    )
    emb_bf = embedding.astype(jnp.bfloat16)[:, _PERM]
    out = run(emb_bf, reservoir_encoded, base_indices.reshape(T))
    return out.reshape(B, L, F)


# speedup vs baseline: 1.0017x; 1.0017x over previous
import jax
import jax.numpy as jnp
import numpy as np
from jax import lax
from jax.experimental import pallas as pl
from jax.experimental.pallas import tpu as pltpu
from jax.experimental.pallas import tpu_sc as plsc

VOCAB, F = 30522, 64
NT, R = 100000, 8
B, L = 4096, 200
T = B * L
NC, NS, LANES = 2, 16, 16
NW = NC * NS
TPW = T // NW
C = 512
BPC = C // L
G = TPW // C
CR = C * R
KPJ = C // LANES

_PERM = np.concatenate(
    [32 * g + np.arange(32).reshape(2, 16).T.reshape(-1) for g in range(F // 32)]
)


def _body(emb, res, bidx, out, bidx2, ids2, fidx2, acc2, fout2, sem_ids,
          sem_add, sem_out):
    cid = lax.axis_index("c")
    sid = lax.axis_index("s")
    wid = sid * NC + cid
    base = wid * TPW

    iota = lax.iota(jnp.int32, LANES)
    zv = jnp.zeros((2 * LANES,), jnp.bfloat16)
    himask = jnp.full((LANES,), -65536, jnp.int32)
    sh16 = jnp.full((LANES,), 16, jnp.int32)

    def front(g, p):
        tok0 = base + g * C
        pltpu.sync_copy(bidx.at[pl.ds(tok0, C)], bidx2.at[p])
        pltpu.async_copy(res.at[bidx2.at[p]], ids2.at[p], sem_ids)

    def wait_ids(p):
        pltpu.make_async_copy(res.at[bidx2.at[p]], ids2.at[p], sem_ids).wait()

    def build(p):
        for k in range(R * KPJ):
            j = k // KPJ
            rows = iota + (k % KPJ) * LANES
            cols = jnp.full((LANES,), j, jnp.int32)
            ids16 = plsc.load_gather(ids2.at[p], [rows, cols])
            fidx2[p, pl.ds(k * LANES, LANES)] = jnp.where(ids16 == 0, -1, ids16)

    def zero_acc(p):
        def zb(t, _):
            for jj in range(F // (2 * LANES)):
                acc2[p, t, pl.ds(jj * 2 * LANES, 2 * LANES)] = zv
            return ()

        lax.fori_loop(0, C, zb, ())

    def widen(p):
        def wb(q, _):
            for par in range(2):
                for grp in range(F // (2 * LANES)):
                    v = acc2[p, 2 * q + par, pl.ds(grp * 2 * LANES, 2 * LANES)]
                    w = plsc.bitcast(v, jnp.int32)
                    lo = plsc.bitcast(lax.shift_left(w, sh16), jnp.float32)
                    hi = plsc.bitcast(lax.bitwise_and(w, himask), jnp.float32)
                    fout2[p, q, pl.ds(par * F + grp * 2 * LANES, LANES)] = lo
                    fout2[p, q, pl.ds(par * F + grp * 2 * LANES + LANES, LANES)] = hi
            return ()

        lax.fori_loop(0, C // 2, wb, ())

    def fire_adds(p):
        descs = []
        for j in range(R):
            idx = plsc.Indices(
                fidx2.at[p].at[pl.ds(j * C, C)], ignored_value=-1
            )
            descs.append(
                pltpu.async_copy(emb.at[idx], acc2.at[p], sem_add, add=True)
            )
        return descs

    def out_issue(g, p):
        row0 = (base + g * C) // 2
        pltpu.async_copy(
            fout2.at[p], out.at[pl.ds(row0, C // 2), :], sem_out
        )

    def wait_out(g, p):
        row0 = (base + g * C) // 2
        pltpu.make_async_copy(
            fout2.at[p], out.at[pl.ds(row0, C // 2), :], sem_out
        ).wait()

    front(0, 0)
    wait_ids(0)
    build(0)
    zero_acc(0)

    def loop_body(i, _):
        for ph in range(2):
            g = 2 * i + ph
            p = ph

            @pl.when(g + 1 < G)
            def _next_front():
                front(g + 1, 1 - p)

            descs = fire_adds(p)

            @pl.when(g >= 3)
            def _reclaim_fout():
                wait_out(g - 3, 1 - p)

            @pl.when(g >= 1)
            def _drain_prev():
                widen(1 - p)
                out_issue(g - 1, 1 - p)

            @pl.when(g + 1 < G)
            def _next_prep():
                wait_ids(1 - p)
                build(1 - p)
                zero_acc(1 - p)

            for d in descs:
                d.wait()
        return ()

    lax.fori_loop(0, G // 2, loop_body, ())
    wait_out(G - 3, 1)
    widen(1)
    out_issue(G - 1, 1)
    wait_out(G - 2, 0)
    wait_out(G - 1, 1)


@jax.jit
def kernel(embedding, reservoir_encoded, base_indices):
    mesh = plsc.VectorSubcoreMesh(core_axis_name="c", subcore_axis_name="s")
    run = pl.kernel(
        _body,
        out_type=jax.ShapeDtypeStruct((T // 2, 2 * F), jnp.float32),
        mesh=mesh,
        compiler_params=pltpu.CompilerParams(
            needs_layout_passes=False, use_tc_tiling_on_sc=False
        ),
        scratch_types=[
            pltpu.VMEM((2, C), jnp.int32),
            pltpu.VMEM((2, C, R), jnp.int32),
            pltpu.VMEM((2, CR), jnp.int32),
            pltpu.VMEM((2, C, F), jnp.bfloat16),
            pltpu.VMEM((2, C // 2, 2 * F), jnp.float32),
            pltpu.SemaphoreType.DMA,
            pltpu.SemaphoreType.DMA,
            pltpu.SemaphoreType.DMA,
        ],
    )
    emb_bf = embedding.astype(jnp.bfloat16)[:, _PERM]
    out = run(emb_bf, reservoir_encoded, base_indices.reshape(T))
    return out.reshape(B, L, F)
